# baseline (speedup 1.0000x reference)
import jax
import jax.numpy as jnp
from jax import lax
from jax.experimental import pallas as pl
from jax.experimental.pallas import tpu as pltpu
from jax.experimental.pallas import tpu_sc as plsc

N = 10000
E = 160000
D = 384
G = 64
NG = D // G
NS = 16
EPT = E // NS
K = 64
NCH = (EPT + K - 1) // K
NCHP = 160
EPAD = NCHP * K
NP = 10240
NRT = 640
NB = 1000
EB = 4000
F32 = jnp.float32


def _k1_body(x_ref, w_ref, as_ref, ad_ref, we_ref, aee_ref, *outs):
    h_refs = outs[:NG]
    s_ref, d_ref, ve_ref = outs[NG:]
    i = pl.program_id(0)
    h = jnp.dot(x_ref[...], w_ref[...], preferred_element_type=F32)
    for g in range(NG):
        h_refs[g][...] = h[:, G * g:G * (g + 1)]
    s_ref[...] = jnp.dot(h, as_ref[0, :], preferred_element_type=F32)[None, None, :]
    d_ref[...] = jnp.dot(h, ad_ref[0, :], preferred_element_type=F32)[None, None, :]

    @pl.when(i == 0)
    def _():
        ve_ref[...] = jnp.dot(we_ref[...], aee_ref[0, :],
                              preferred_element_type=F32)[None, :]


def _k1(node_emb, W, att_src, att_dst, W_e, att_edge):
    nblk = N // NB
    return pl.pallas_call(
        _k1_body,
        grid=(nblk,),
        in_specs=[
            pl.BlockSpec((NB, D), lambda i: (i, 0)),
            pl.BlockSpec((D, D), lambda i: (0, 0)),
            pl.BlockSpec((1, D), lambda i: (0, 0)),
            pl.BlockSpec((1, D), lambda i: (0, 0)),
            pl.BlockSpec((D, D), lambda i: (0, 0)),
            pl.BlockSpec((1, D), lambda i: (0, 0)),
        ],
        out_specs=(
            [pl.BlockSpec((NB, G), lambda i: (i, 0)) for _ in range(NG)]
            + [pl.BlockSpec((1, 1, NB), lambda i: (i, 0, 0)),
               pl.BlockSpec((1, 1, NB), lambda i: (i, 0, 0)),
               pl.BlockSpec((1, D), lambda i: (0, 0))]
        ),
        out_shape=(
            [jax.ShapeDtypeStruct((N, G), F32) for _ in range(NG)]
            + [jax.ShapeDtypeStruct((nblk, 1, NB), F32),
               jax.ShapeDtypeStruct((nblk, 1, NB), F32),
               jax.ShapeDtypeStruct((1, D), F32)]
        ),
    )(node_emb, W, att_src.reshape(1, D), att_dst.reshape(1, D),
      W_e, att_edge.reshape(1, D))


def _k2_body(ea_ref, ve_ref, ae_ref):
    ae_ref[...] = jnp.sum(ea_ref[...] * ve_ref[...], axis=1)[None, None, :]


def _k2(edge_attr, ve):
    eblk = E // EB
    return pl.pallas_call(
        _k2_body,
        grid=(eblk,),
        in_specs=[
            pl.BlockSpec((EB, D), lambda i: (i, 0)),
            pl.BlockSpec((1, D), lambda i: (0, 0)),
        ],
        out_specs=pl.BlockSpec((1, 1, EB), lambda i: (i, 0, 0)),
        out_shape=jax.ShapeDtypeStruct((eblk, 1, EB), F32),
    )(edge_attr, ve)


def _sc_body(*refs):
    (srcp, dstp, aep, asrc_h, adst_h) = refs[:5]
    h_hs = refs[5:5 + NG]
    out_hs = refs[5 + NG:5 + 2 * NG]
    (asrc_v, adst_v, w2_v, degt_v, saet_v, st_v, rows0, rows1,
     dst2_v, srcr, aer, cmax_v, el_v, rr_v, mx_v, iota2_v,
     acc_g, degg_g, saeg_g, sg_g, cmax_g,
     semA, semG0, semG1, semS0, semS1) = refs[5 + 2 * NG:]

    cid = lax.axis_index("c")
    sid = lax.axis_index("s")
    zeros16 = jnp.zeros((16,), F32)
    ones16 = jnp.ones((16,), F32)
    nsub = EPT // 16

    def _zero_rows0():
        def _z(r, _):
            for t in range(G // 16):
                rows0[r, pl.ds(16 * t, 16)] = zeros16
            return 0
        lax.fori_loop(0, K, _z, 0)

    _zero_rows0()

    def _z_tab(c, _):
        degt_v[c, pl.ds(0, 16)] = zeros16
        saet_v[c, pl.ds(0, 16)] = zeros16
        st_v[c, pl.ds(0, 16)] = zeros16
        return 0
    lax.fori_loop(0, NRT, _z_tab, 0)
    for t in range((EPAD - EPT) // 16):
        w2_v[pl.ds(EPT + 16 * t, 16)] = zeros16

    def _bi(c, _):
        iota2_v[c // 8, pl.ds(16 * (c % 8), 16)] = (
            lax.broadcasted_iota(jnp.int32, (16,), 0) + 16 * c)
        return 0
    lax.fori_loop(0, NRT // 16, _bi, 0)

    @pl.when(sid == 0)
    def _():
        pltpu.sync_copy(degt_v, degg_g)
        pltpu.sync_copy(saet_v, saeg_g)
        pltpu.sync_copy(st_v, sg_g)

    pltpu.sync_copy(asrc_h, asrc_v)
    pltpu.sync_copy(adst_h, adst_v)
    pltpu.sync_copy(dstp.at[sid], dst2_v)
    plsc.subcore_barrier()

    mx_v[...] = jnp.full((16,), -3e38, F32)

    def _fetch1(g, slot):
        pltpu.async_copy(srcp.at[sid, pl.ds(4 * g, 4)],
                         srcr.at[pl.ds(4 * slot, 4)], semA)
        pltpu.async_copy(aep.at[sid, pl.ds(4 * g, 4)],
                         aer.at[pl.ds(4 * slot, 4)], semA)

    def _wait1(slot):
        pltpu.make_async_copy(srcp.at[0, pl.ds(0, 4)],
                              srcr.at[pl.ds(4 * slot, 4)], semA).wait()
        pltpu.make_async_copy(aep.at[0, pl.ds(0, 4)],
                              aer.at[pl.ds(4 * slot, 4)], semA).wait()

    def _proc1(g, slot):
        def _sub(k, _):
            c = 16 * g + k

            @pl.when(c < nsub)
            def _():
                lr = 4 * slot + k // 4
                col = (k % 4) * 16
                jr = c // 4
                jc = (c % 4) * 16
                s16 = srcr[lr, pl.ds(col, 16)]
                d16 = dst2_v[jr, pl.ds(jc, 16)]
                a1 = plsc.load_gather(asrc_v, [s16])
                a2 = plsc.load_gather(adst_v, [d16])
                ae16 = aer[lr, pl.ds(col, 16)]
                al = a1 + a2 + ae16
                al = jnp.where(al >= 0.0, al, al * 0.2)
                w2_v[pl.ds(16 * c, 16)] = al
                dr16 = lax.shift_right_logical(d16, 4)
                dc16 = lax.bitwise_and(d16, jnp.int32(15))
                plsc.addupdate_scatter(degt_v, [dr16, dc16], ones16)
                plsc.addupdate_scatter(saet_v, [dr16, dc16], ae16)
                mx_v[...] = jnp.maximum(mx_v[...], al)
            return 0
        lax.fori_loop(0, 16, _sub, 0)

    ngrp = NCHP // 4
    _fetch1(0, 0)
    _fetch1(1, 1)

    def _p1(gg, _):
        g0 = 2 * gg
        _wait1(0)
        _proc1(g0, 0)

        @pl.when(g0 + 2 < ngrp)
        def _():
            _fetch1(g0 + 2, 0)
        _wait1(1)
        _proc1(g0 + 1, 1)

        @pl.when(g0 + 3 < ngrp)
        def _():
            _fetch1(g0 + 3, 1)
        return 0
    lax.fori_loop(0, ngrp // 2, _p1, 0)

    el_v[...] = jnp.full((16,), jnp.max(mx_v[...]), F32)
    pltpu.sync_copy(el_v, cmax_g.at[sid])

    for r in range(NRT // 128):
        sl = pl.ds(128 * r, 128)
        pltpu.sync_copy(degt_v.at[sl], degg_g.at[iota2_v.at[r]], add=True)
        pltpu.sync_copy(saet_v.at[sl], saeg_g.at[iota2_v.at[r]], add=True)
    plsc.subcore_barrier()

    pltpu.sync_copy(cmax_g, cmax_v)
    cm = jnp.full((16,), -3e38, F32)
    for i in range(NS):
        cm = jnp.maximum(cm, cmax_v[i, pl.ds(0, 16)])
    cglob = jnp.max(cm)

    def _p2(c, _):
        jr = c // 4
        jc = (c % 4) * 16
        d16 = dst2_v[jr, pl.ds(jc, 16)]
        w16 = jnp.exp(w2_v[pl.ds(16 * c, 16)] - cglob)
        w2_v[pl.ds(16 * c, 16)] = w16
        dr16 = lax.shift_right_logical(d16, 4)
        dc16 = lax.bitwise_and(d16, jnp.int32(15))
        plsc.addupdate_scatter(st_v, [dr16, dc16], w16)
        return 0
    lax.fori_loop(0, nsub, _p2, 0)

    for r in range(NRT // 128):
        pltpu.sync_copy(st_v.at[pl.ds(128 * r, 128)],
                        sg_g.at[iota2_v.at[r]], add=True)
    plsc.subcore_barrier()

    pltpu.sync_copy(degg_g, degt_v)
    pltpu.sync_copy(saeg_g, saet_v)
    pltpu.sync_copy(sg_g, st_v)
    plsc.subcore_barrier()

    def _pre(q, _):
        nc = 40 * sid + q
        b = 16 * nc
        deg16 = degt_v[nc, pl.ds(0, 16)]
        sae16 = saet_v[nc, pl.ds(0, 16)]
        s16 = st_v[nc, pl.ds(0, 16)]
        al = (asrc_v[pl.ds(b, 16)] + adst_v[pl.ds(b, 16)]
              + sae16 / jnp.maximum(deg16, 1.0))
        al = jnp.where(al >= 0.0, al, al * 0.2)
        el16 = jnp.exp(al - cglob)
        r16 = 1.0 / (s16 + el16 + 1e-16)
        asrc_v[pl.ds(b, 16)] = el16
        adst_v[pl.ds(b, 16)] = r16
        return 0
    lax.fori_loop(0, NP // (16 * NS), _pre, 0)

    def _group(h_h, out_h, first):
        if first:
            _zero_rows0()

            def _z_acc(t, _):
                blk = sid + NS * t
                pltpu.sync_copy(rows0, acc_g.at[pl.ds(K * blk, K)])
                return 0
            lax.fori_loop(0, NP // (NS * K), _z_acc, 0)
            plsc.subcore_barrier()

        def _fetchrow(j, slot):
            pltpu.sync_copy(srcp.at[sid, j], srcr.at[slot])

        def _gather(buf, sem, slot):
            pltpu.async_copy(h_h.at[srcr.at[slot]], buf, sem)

        def _wait_g(buf, sem, slot):
            pltpu.make_async_copy(h_h.at[srcr.at[slot]], buf, sem).wait()

        def _proc4(buf, j):
            @plsc.parallel_loop(0, K, step=1, unroll=4)
            def _prow(u):
                wv = plsc.load_gather(
                    w2_v, [jnp.full((16,), K * j + u, jnp.int32)])
                for t in range(G // 16):
                    sl = pl.ds(16 * t, 16)
                    buf[u, sl] = buf[u, sl] * wv

        def _scat(buf, sem, j):
            pltpu.async_copy(buf, acc_g.at[dst2_v.at[j]], sem, add=True)

        def _wait_s(buf, sem):
            pltpu.make_async_copy(buf, acc_g.at[dst2_v.at[0]], sem).wait()

        _fetchrow(0, 0)
        _gather(rows0, semG0, 0)
        _fetchrow(1, 1)
        _gather(rows1, semG1, 1)

        def _p4(kk, _):
            j0 = 2 * kk
            _wait_g(rows0, semG0, 0)
            _proc4(rows0, j0)
            _scat(rows0, semS0, j0)
            _wait_g(rows1, semG1, 1)
            _proc4(rows1, j0 + 1)
            _scat(rows1, semS1, j0 + 1)

            @pl.when(j0 + 2 < NCH)
            def _():
                _wait_s(rows0, semS0)
                _fetchrow(j0 + 2, 0)
                _gather(rows0, semG0, 0)

            @pl.when(j0 + 3 < NCH)
            def _():
                _wait_s(rows1, semS1)
                _fetchrow(j0 + 3, 1)
                _gather(rows1, semG1, 1)
            return 0
        lax.fori_loop(0, NCH // 2, _p4, 0)
        _wait_g(rows0, semG0, 0)
        _proc4(rows0, NCH - 1)
        _scat(rows0, semS0, NCH - 1)
        _wait_s(rows0, semS0)
        _wait_s(rows1, semS1)

        iota16 = lax.broadcasted_iota(jnp.int32, (16,), 0)

        def _self_idx(slot, rb):
            for t in range(4):
                srcr[slot, pl.ds(16 * t, 16)] = iota16 + (rb + 16 * t)

        def _proc4b(buf, rb):
            @plsc.parallel_loop(0, K, step=1, unroll=4)
            def _prow(u):
                wv = plsc.load_gather(
                    asrc_v, [jnp.full((16,), rb + u, jnp.int32)])
                for t in range(G // 16):
                    sl = pl.ds(16 * t, 16)
                    buf[u, sl] = buf[u, sl] * wv

        rb0 = 640 * sid
        pltpu.async_copy(h_h.at[pl.ds(rb0, K)], rows0, semG0)
        pltpu.async_copy(h_h.at[pl.ds(rb0 + K, K)], rows1, semG1)

        def _p4b(bb, _):
            b0 = 2 * bb
            rbA = rb0 + K * b0
            pltpu.make_async_copy(h_h.at[pl.ds(0, K)], rows0, semG0).wait()
            _proc4b(rows0, rbA)
            _self_idx(0, rbA)
            pltpu.async_copy(rows0, acc_g.at[srcr.at[0]], semS0, add=True)

            @pl.when(b0 + 2 < 10)
            def _():
                _wait_s(rows0, semS0)
                pltpu.async_copy(h_h.at[pl.ds(rbA + 2 * K, K)], rows0, semG0)
            pltpu.make_async_copy(h_h.at[pl.ds(0, K)], rows1, semG1).wait()
            _proc4b(rows1, rbA + K)
            _self_idx(1, rbA + K)
            pltpu.async_copy(rows1, acc_g.at[srcr.at[1]], semS1, add=True)

            @pl.when(b0 + 3 < 10)
            def _():
                _wait_s(rows1, semS1)
                pltpu.async_copy(h_h.at[pl.ds(rbA + 3 * K, K)], rows1, semG1)
            return 0
        lax.fori_loop(0, 5, _p4b, 0)
        _wait_s(rows0, semS0)
        _wait_s(rows1, semS1)
        plsc.subcore_barrier()

        _zero_rows0()

        def _p5(bb, _):
            rb = 640 * sid + K * bb
            pltpu.sync_copy(acc_g.at[pl.ds(rb, K)], rows1)
            pltpu.sync_copy(rows0, acc_g.at[pl.ds(rb, K)])

            @plsc.parallel_loop(0, K, step=1, unroll=4)
            def _prow(u):
                rv = plsc.load_gather(
                    adst_v, [jnp.full((16,), rb + u, jnp.int32)])
                for t2 in range(G // 16):
                    sl = pl.ds(16 * t2, 16)
                    rows1[u, sl] = rows1[u, sl] * rv
            pltpu.sync_copy(rows1, out_h.at[pl.ds(rb, K)])
            return 0
        lax.fori_loop(0, NP // (NS * K), _p5, 0)
        plsc.subcore_barrier()

    for g in range(NG // 2):
        @pl.when(cid == 0)
        def _(g=g):
            _group(h_hs[g], out_hs[g], g == 0)

        @pl.when(cid == 1)
        def _(g=g):
            _group(h_hs[NG // 2 + g], out_hs[NG // 2 + g], g == 0)


def _k3(srcp, dstp, aep, asrc, adst, hs):
    mesh = plsc.VectorSubcoreMesh(core_axis_name="c", subcore_axis_name="s",
                                  num_cores=2, num_subcores=NS)
    f = pl.kernel(
        _sc_body,
        out_type=[jax.ShapeDtypeStruct((NP, G), F32) for _ in range(NG)],
        mesh=mesh,
        compiler_params=pltpu.CompilerParams(needs_layout_passes=False,
                                             use_tc_tiling_on_sc=False),
        scratch_types=[
            pltpu.VMEM((NP,), F32),
            pltpu.VMEM((NP,), F32),
            pltpu.VMEM((EPAD,), F32),
            pltpu.VMEM((NRT, 16), F32),
            pltpu.VMEM((NRT, 16), F32),
            pltpu.VMEM((NRT, 16), F32),
            pltpu.VMEM((K, G), F32),
            pltpu.VMEM((K, G), F32),
            pltpu.VMEM((NCHP, K), jnp.int32),
            pltpu.VMEM((8, K), jnp.int32),
            pltpu.VMEM((8, K), F32),
            pltpu.VMEM((NS, 16), F32),
            pltpu.VMEM((16,), F32),
            pltpu.VMEM((16,), F32),
            pltpu.VMEM((16,), F32),
            pltpu.VMEM((5, 128), jnp.int32),
            pltpu.VMEM_SHARED((NP, G), F32),
            pltpu.VMEM_SHARED((NRT, 16), F32),
            pltpu.VMEM_SHARED((NRT, 16), F32),
            pltpu.VMEM_SHARED((NRT, 16), F32),
            pltpu.VMEM_SHARED((NS, 16), F32),
            pltpu.SemaphoreType.DMA,
            pltpu.SemaphoreType.DMA,
            pltpu.SemaphoreType.DMA,
            pltpu.SemaphoreType.DMA,
            pltpu.SemaphoreType.DMA,
        ],
    )
    return f(srcp, dstp, aep, asrc, adst, *hs)


def _k4_body(*refs):
    o_refs = refs[:NG]
    q_ref, b_ref = refs[NG:NG + 2]
    out_ref = refs[NG + 2]
    bestv, brow = refs[NG + 3:]

    i = pl.program_id(0)
    nblk = pl.num_programs(0)
    q = q_ref[...]
    qn = q / jnp.maximum(jnp.sqrt(jnp.sum(q * q)), 1e-8)
    obs = []
    dots = jnp.zeros((NB, 1), F32)
    nn = jnp.zeros((NB, 1), F32)
    for g in range(NG):
        ob = o_refs[g][...] + b_ref[:, G * g:G * (g + 1)]
        obs.append(ob)
        dots = dots + jnp.sum(ob * qn[:, G * g:G * (g + 1)], axis=1,
                              keepdims=True)
        nn = nn + jnp.sum(ob * ob, axis=1, keepdims=True)
    sim = dots / jnp.maximum(jnp.sqrt(nn), 1e-8)
    m = jnp.max(sim)
    iota = lax.broadcasted_iota(jnp.int32, sim.shape, 0)
    il = jnp.min(jnp.where(sim == m, iota, jnp.int32(2**30)))

    @pl.when((i == 0) | (m > bestv[0]))
    def _():
        bestv[0] = m
        for g in range(NG):
            sel = jnp.where(iota == il, obs[g], 0.0)
            brow[:, pl.ds(G * g, G)] = jnp.sum(sel, axis=0, keepdims=True)

    @pl.when(i == nblk - 1)
    def _():
        out_ref[...] = brow[...]


def _k4(outs, q, bias):
    nblk = N // NB
    return pl.pallas_call(
        _k4_body,
        grid=(nblk,),
        in_specs=(
            [pl.BlockSpec((NB, G), lambda i: (i, 0)) for _ in range(NG)]
            + [pl.BlockSpec((1, D), lambda i: (0, 0)),
               pl.BlockSpec((1, D), lambda i: (0, 0))]
        ),
        out_specs=pl.BlockSpec((1, D), lambda i: (0, 0)),
        out_shape=jax.ShapeDtypeStruct((1, D), F32),
        scratch_shapes=[
            pltpu.SMEM((1,), F32),
            pltpu.VMEM((1, D), F32),
        ],
    )(*outs, q.reshape(1, D), bias.reshape(1, D))


def kernel(query_emb, edge_index, edge_attr, node_emb, W, att_src, att_dst,
           W_e, att_edge, bias):
    src = edge_index[0].reshape(NS, EPT)
    dst = edge_index[1].reshape(NS, EPT)
    srcp = jnp.pad(src, ((0, 0), (0, EPAD - EPT))).reshape(NS, NCHP, K)
    dstp = jnp.pad(dst, ((0, 0), (0, EPAD - EPT))).reshape(NS, NCHP, K)

    k1 = _k1(node_emb, W, att_src, att_dst, W_e, att_edge)
    hs = [jnp.pad(h, ((0, NP - N), (0, 0))) for h in k1[:NG]]
    asrc2, adst2, ve = k1[NG:]
    asrcp = jnp.pad(asrc2.reshape(N), (0, NP - N))
    adstp = jnp.pad(adst2.reshape(N), (0, NP - N))
    ae2 = _k2(edge_attr, ve)
    aep = jnp.pad(ae2.reshape(NS, EPT), ((0, 0), (0, EPAD - EPT))
                  ).reshape(NS, NCHP, K)
    outs = _k3(srcp, dstp, aep, asrcp, adstp, hs)
    res = _k4([o[:N] for o in outs], query_emb, bias)
    return res.reshape(D)


# speedup vs baseline: 1.0456x; 1.0456x over previous
import jax
import jax.numpy as jnp
from jax import lax
from jax.experimental import pallas as pl
from jax.experimental.pallas import tpu as pltpu
from jax.experimental.pallas import tpu_sc as plsc

N = 10000
E = 160000
D = 384
G = 64
NG = D // G
NS = 16
EPT = E // NS
K = 64
NCH = (EPT + K - 1) // K
NCHP = 160
EPAD = NCHP * K
NP = 10240
NRT = 640
NB = 1000
EB = 4000
F32 = jnp.float32


def _k1_body(x_ref, w_ref, as_ref, ad_ref, we_ref, aee_ref, *outs):
    h_refs = outs[:NG]
    s_ref, d_ref, ve_ref = outs[NG:]
    i = pl.program_id(0)
    h = jnp.dot(x_ref[...], w_ref[...], preferred_element_type=F32)
    for g in range(NG):
        h_refs[g][...] = h[:, G * g:G * (g + 1)]
    s_ref[...] = jnp.dot(h, as_ref[0, :], preferred_element_type=F32)[None, None, :]
    d_ref[...] = jnp.dot(h, ad_ref[0, :], preferred_element_type=F32)[None, None, :]

    @pl.when(i == 0)
    def _():
        ve_ref[...] = jnp.dot(we_ref[...], aee_ref[0, :],
                              preferred_element_type=F32)[None, :]


def _k1(node_emb, W, att_src, att_dst, W_e, att_edge):
    nblk = N // NB
    return pl.pallas_call(
        _k1_body,
        grid=(nblk,),
        in_specs=[
            pl.BlockSpec((NB, D), lambda i: (i, 0)),
            pl.BlockSpec((D, D), lambda i: (0, 0)),
            pl.BlockSpec((1, D), lambda i: (0, 0)),
            pl.BlockSpec((1, D), lambda i: (0, 0)),
            pl.BlockSpec((D, D), lambda i: (0, 0)),
            pl.BlockSpec((1, D), lambda i: (0, 0)),
        ],
        out_specs=(
            [pl.BlockSpec((NB, G), lambda i: (i, 0)) for _ in range(NG)]
            + [pl.BlockSpec((1, 1, NB), lambda i: (i, 0, 0)),
               pl.BlockSpec((1, 1, NB), lambda i: (i, 0, 0)),
               pl.BlockSpec((1, D), lambda i: (0, 0))]
        ),
        out_shape=(
            [jax.ShapeDtypeStruct((N, G), F32) for _ in range(NG)]
            + [jax.ShapeDtypeStruct((nblk, 1, NB), F32),
               jax.ShapeDtypeStruct((nblk, 1, NB), F32),
               jax.ShapeDtypeStruct((1, D), F32)]
        ),
    )(node_emb, W, att_src.reshape(1, D), att_dst.reshape(1, D),
      W_e, att_edge.reshape(1, D))


def _k2_body(ea_ref, ve_ref, ae_ref):
    ae_ref[...] = jnp.sum(ea_ref[...] * ve_ref[...], axis=1)[None, None, :]


def _k2(edge_attr, ve):
    eblk = E // EB
    return pl.pallas_call(
        _k2_body,
        grid=(eblk,),
        in_specs=[
            pl.BlockSpec((EB, D), lambda i: (i, 0)),
            pl.BlockSpec((1, D), lambda i: (0, 0)),
        ],
        out_specs=pl.BlockSpec((1, 1, EB), lambda i: (i, 0, 0)),
        out_shape=jax.ShapeDtypeStruct((eblk, 1, EB), F32),
    )(edge_attr, ve)


def _sc_body(*refs):
    (srcp, dstp, aep, asrc_h, adst_h) = refs[:5]
    h_hs = refs[5:5 + NG]
    out_hs = refs[5 + NG:5 + 2 * NG]
    (asrc_v, adst_v, w2_v, degt_v, saet_v, st_v, rows0, rows1, rows2,
     dst2_v, srcr, aer, cmax_v, el_v, rr_v, mx_v, iota2_v,
     acc_g, degg_g, saeg_g, sg_g, cmax_g,
     semA, semG0, semG1, semG2, semS0, semS1, semS2) = refs[5 + 2 * NG:]

    cid = lax.axis_index("c")
    sid = lax.axis_index("s")
    zeros16 = jnp.zeros((16,), F32)
    ones16 = jnp.ones((16,), F32)
    nsub = EPT // 16

    def _zero_rows0():
        def _z(r, _):
            for t in range(G // 16):
                rows0[r, pl.ds(16 * t, 16)] = zeros16
            return 0
        lax.fori_loop(0, K, _z, 0)

    _zero_rows0()

    def _z_tab(c, _):
        degt_v[c, pl.ds(0, 16)] = zeros16
        saet_v[c, pl.ds(0, 16)] = zeros16
        st_v[c, pl.ds(0, 16)] = zeros16
        return 0
    lax.fori_loop(0, NRT, _z_tab, 0)
    for t in range((EPAD - EPT) // 16):
        w2_v[pl.ds(EPT + 16 * t, 16)] = zeros16

    def _bi(c, _):
        iota2_v[c // 8, pl.ds(16 * (c % 8), 16)] = (
            lax.broadcasted_iota(jnp.int32, (16,), 0) + 16 * c)
        return 0
    lax.fori_loop(0, NRT // 16, _bi, 0)

    @pl.when(sid == 0)
    def _():
        pltpu.sync_copy(degt_v, degg_g)
        pltpu.sync_copy(saet_v, saeg_g)
        pltpu.sync_copy(st_v, sg_g)

    pltpu.sync_copy(asrc_h, asrc_v)
    pltpu.sync_copy(adst_h, adst_v)
    pltpu.sync_copy(dstp.at[sid], dst2_v)
    plsc.subcore_barrier()

    mx_v[...] = jnp.full((16,), -3e38, F32)

    def _fetch1(g, slot):
        pltpu.async_copy(srcp.at[sid, pl.ds(4 * g, 4)],
                         srcr.at[pl.ds(4 * slot, 4)], semA)
        pltpu.async_copy(aep.at[sid, pl.ds(4 * g, 4)],
                         aer.at[pl.ds(4 * slot, 4)], semA)

    def _wait1(slot):
        pltpu.make_async_copy(srcp.at[0, pl.ds(0, 4)],
                              srcr.at[pl.ds(4 * slot, 4)], semA).wait()
        pltpu.make_async_copy(aep.at[0, pl.ds(0, 4)],
                              aer.at[pl.ds(4 * slot, 4)], semA).wait()

    def _proc1(g, slot):
        def _sub(k, _):
            c = 16 * g + k

            @pl.when(c < nsub)
            def _():
                lr = 4 * slot + k // 4
                col = (k % 4) * 16
                jr = c // 4
                jc = (c % 4) * 16
                s16 = srcr[lr, pl.ds(col, 16)]
                d16 = dst2_v[jr, pl.ds(jc, 16)]
                a1 = plsc.load_gather(asrc_v, [s16])
                a2 = plsc.load_gather(adst_v, [d16])
                ae16 = aer[lr, pl.ds(col, 16)]
                al = a1 + a2 + ae16
                al = jnp.where(al >= 0.0, al, al * 0.2)
                w2_v[pl.ds(16 * c, 16)] = al
                dr16 = lax.shift_right_logical(d16, 4)
                dc16 = lax.bitwise_and(d16, jnp.int32(15))
                plsc.addupdate_scatter(degt_v, [dr16, dc16], ones16)
                plsc.addupdate_scatter(saet_v, [dr16, dc16], ae16)
                mx_v[...] = jnp.maximum(mx_v[...], al)
            return 0
        lax.fori_loop(0, 16, _sub, 0)

    ngrp = NCHP // 4
    _fetch1(0, 0)
    _fetch1(1, 1)

    def _p1(gg, _):
        g0 = 2 * gg
        _wait1(0)
        _proc1(g0, 0)

        @pl.when(g0 + 2 < ngrp)
        def _():
            _fetch1(g0 + 2, 0)
        _wait1(1)
        _proc1(g0 + 1, 1)

        @pl.when(g0 + 3 < ngrp)
        def _():
            _fetch1(g0 + 3, 1)
        return 0
    lax.fori_loop(0, ngrp // 2, _p1, 0)

    el_v[...] = jnp.full((16,), jnp.max(mx_v[...]), F32)
    pltpu.sync_copy(el_v, cmax_g.at[sid])

    for r in range(NRT // 128):
        sl = pl.ds(128 * r, 128)
        pltpu.sync_copy(degt_v.at[sl], degg_g.at[iota2_v.at[r]], add=True)
        pltpu.sync_copy(saet_v.at[sl], saeg_g.at[iota2_v.at[r]], add=True)
    plsc.subcore_barrier()

    pltpu.sync_copy(cmax_g, cmax_v)
    cm = jnp.full((16,), -3e38, F32)
    for i in range(NS):
        cm = jnp.maximum(cm, cmax_v[i, pl.ds(0, 16)])
    cglob = jnp.max(cm)

    def _p2(c, _):
        jr = c // 4
        jc = (c % 4) * 16
        d16 = dst2_v[jr, pl.ds(jc, 16)]
        w16 = jnp.exp(w2_v[pl.ds(16 * c, 16)] - cglob)
        w2_v[pl.ds(16 * c, 16)] = w16
        dr16 = lax.shift_right_logical(d16, 4)
        dc16 = lax.bitwise_and(d16, jnp.int32(15))
        plsc.addupdate_scatter(st_v, [dr16, dc16], w16)
        return 0
    lax.fori_loop(0, nsub, _p2, 0)

    for r in range(NRT // 128):
        pltpu.sync_copy(st_v.at[pl.ds(128 * r, 128)],
                        sg_g.at[iota2_v.at[r]], add=True)
    plsc.subcore_barrier()

    pltpu.sync_copy(degg_g, degt_v)
    pltpu.sync_copy(saeg_g, saet_v)
    pltpu.sync_copy(sg_g, st_v)
    plsc.subcore_barrier()

    def _pre(q, _):
        nc = 40 * sid + q
        b = 16 * nc
        deg16 = degt_v[nc, pl.ds(0, 16)]
        sae16 = saet_v[nc, pl.ds(0, 16)]
        s16 = st_v[nc, pl.ds(0, 16)]
        al = (asrc_v[pl.ds(b, 16)] + adst_v[pl.ds(b, 16)]
              + sae16 / jnp.maximum(deg16, 1.0))
        al = jnp.where(al >= 0.0, al, al * 0.2)
        el16 = jnp.exp(al - cglob)
        r16 = 1.0 / (s16 + el16 + 1e-16)
        asrc_v[pl.ds(b, 16)] = el16
        adst_v[pl.ds(b, 16)] = r16
        return 0
    lax.fori_loop(0, NP // (16 * NS), _pre, 0)

    def _group(h_h, out_h, first):
        if first:
            _zero_rows0()

            def _z_acc(t, _):
                blk = sid + NS * t
                pltpu.sync_copy(rows0, acc_g.at[pl.ds(K * blk, K)])
                return 0
            lax.fori_loop(0, NP // (NS * K), _z_acc, 0)
            plsc.subcore_barrier()

        def _fetchrow(j, slot):
            pltpu.sync_copy(srcp.at[sid, j], srcr.at[slot])

        def _gather(buf, sem, slot):
            pltpu.async_copy(h_h.at[srcr.at[slot]], buf, sem)

        def _wait_g(buf, sem, slot):
            pltpu.make_async_copy(h_h.at[srcr.at[slot]], buf, sem).wait()

        def _proc4(buf, j):
            @plsc.parallel_loop(0, K, step=1, unroll=4)
            def _prow(u):
                wv = plsc.load_gather(
                    w2_v, [jnp.full((16,), K * j + u, jnp.int32)])
                for t in range(G // 16):
                    sl = pl.ds(16 * t, 16)
                    buf[u, sl] = buf[u, sl] * wv

        def _scat(buf, sem, j):
            pltpu.async_copy(buf, acc_g.at[dst2_v.at[j]], sem, add=True)

        def _wait_s(buf, sem):
            pltpu.make_async_copy(buf, acc_g.at[dst2_v.at[0]], sem).wait()

        bufs = (rows0, rows1, rows2)
        semsG = (semG0, semG1, semG2)
        semsS = (semS0, semS1, semS2)
        for p in range(3):
            _fetchrow(p, p)
            _gather(bufs[p], semsG[p], p)

        def _p4(kk, _):
            for p in range(3):
                j = 3 * kk + p
                _wait_g(bufs[p], semsG[p], p)
                _proc4(bufs[p], j)
                _scat(bufs[p], semsS[p], j)

                @pl.when(j + 3 < NCH)
                def _(j=j, p=p):
                    _wait_s(bufs[p], semsS[p])
                    _fetchrow(j + 3, p)
                    _gather(bufs[p], semsG[p], p)
            return 0
        lax.fori_loop(0, NCH // 3, _p4, 0)
        _wait_g(rows0, semG0, 0)
        _proc4(rows0, NCH - 1)
        _scat(rows0, semS0, NCH - 1)
        _wait_s(rows0, semS0)
        _wait_s(rows1, semS1)
        _wait_s(rows2, semS2)

        iota16 = lax.broadcasted_iota(jnp.int32, (16,), 0)

        def _self_idx(slot, rb):
            for t in range(4):
                srcr[slot, pl.ds(16 * t, 16)] = iota16 + (rb + 16 * t)

        def _proc4b(buf, rb):
            @plsc.parallel_loop(0, K, step=1, unroll=4)
            def _prow(u):
                wv = plsc.load_gather(
                    asrc_v, [jnp.full((16,), rb + u, jnp.int32)])
                for t in range(G // 16):
                    sl = pl.ds(16 * t, 16)
                    buf[u, sl] = buf[u, sl] * wv

        rb0 = 640 * sid
        pltpu.async_copy(h_h.at[pl.ds(rb0, K)], rows0, semG0)
        pltpu.async_copy(h_h.at[pl.ds(rb0 + K, K)], rows1, semG1)

        def _p4b(bb, _):
            b0 = 2 * bb
            rbA = rb0 + K * b0
            pltpu.make_async_copy(h_h.at[pl.ds(0, K)], rows0, semG0).wait()
            _proc4b(rows0, rbA)
            _self_idx(0, rbA)
            pltpu.async_copy(rows0, acc_g.at[srcr.at[0]], semS0, add=True)

            @pl.when(b0 + 2 < 10)
            def _():
                _wait_s(rows0, semS0)
                pltpu.async_copy(h_h.at[pl.ds(rbA + 2 * K, K)], rows0, semG0)
            pltpu.make_async_copy(h_h.at[pl.ds(0, K)], rows1, semG1).wait()
            _proc4b(rows1, rbA + K)
            _self_idx(1, rbA + K)
            pltpu.async_copy(rows1, acc_g.at[srcr.at[1]], semS1, add=True)

            @pl.when(b0 + 3 < 10)
            def _():
                _wait_s(rows1, semS1)
                pltpu.async_copy(h_h.at[pl.ds(rbA + 3 * K, K)], rows1, semG1)
            return 0
        lax.fori_loop(0, 5, _p4b, 0)
        _wait_s(rows0, semS0)
        _wait_s(rows1, semS1)
        plsc.subcore_barrier()

        _zero_rows0()

        def _p5(bb, _):
            rb = 640 * sid + K * bb
            pltpu.sync_copy(acc_g.at[pl.ds(rb, K)], rows1)
            pltpu.sync_copy(rows0, acc_g.at[pl.ds(rb, K)])

            @plsc.parallel_loop(0, K, step=1, unroll=4)
            def _prow(u):
                rv = plsc.load_gather(
                    adst_v, [jnp.full((16,), rb + u, jnp.int32)])
                for t2 in range(G // 16):
                    sl = pl.ds(16 * t2, 16)
                    rows1[u, sl] = rows1[u, sl] * rv
            pltpu.sync_copy(rows1, out_h.at[pl.ds(rb, K)])
            return 0
        lax.fori_loop(0, NP // (NS * K), _p5, 0)
        plsc.subcore_barrier()

    for g in range(NG // 2):
        @pl.when(cid == 0)
        def _(g=g):
            _group(h_hs[g], out_hs[g], g == 0)

        @pl.when(cid == 1)
        def _(g=g):
            _group(h_hs[NG // 2 + g], out_hs[NG // 2 + g], g == 0)


def _k3(srcp, dstp, aep, asrc, adst, hs):
    mesh = plsc.VectorSubcoreMesh(core_axis_name="c", subcore_axis_name="s",
                                  num_cores=2, num_subcores=NS)
    f = pl.kernel(
        _sc_body,
        out_type=[jax.ShapeDtypeStruct((NP, G), F32) for _ in range(NG)],
        mesh=mesh,
        compiler_params=pltpu.CompilerParams(needs_layout_passes=False,
                                             use_tc_tiling_on_sc=False),
        scratch_types=[
            pltpu.VMEM((NP,), F32),
            pltpu.VMEM((NP,), F32),
            pltpu.VMEM((EPAD,), F32),
            pltpu.VMEM((NRT, 16), F32),
            pltpu.VMEM((NRT, 16), F32),
            pltpu.VMEM((NRT, 16), F32),
            pltpu.VMEM((K, G), F32),
            pltpu.VMEM((K, G), F32),
            pltpu.VMEM((K, G), F32),
            pltpu.VMEM((NCHP, K), jnp.int32),
            pltpu.VMEM((8, K), jnp.int32),
            pltpu.VMEM((8, K), F32),
            pltpu.VMEM((NS, 16), F32),
            pltpu.VMEM((16,), F32),
            pltpu.VMEM((16,), F32),
            pltpu.VMEM((16,), F32),
            pltpu.VMEM((5, 128), jnp.int32),
            pltpu.VMEM_SHARED((NP, G), F32),
            pltpu.VMEM_SHARED((NRT, 16), F32),
            pltpu.VMEM_SHARED((NRT, 16), F32),
            pltpu.VMEM_SHARED((NRT, 16), F32),
            pltpu.VMEM_SHARED((NS, 16), F32),
            pltpu.SemaphoreType.DMA,
            pltpu.SemaphoreType.DMA,
            pltpu.SemaphoreType.DMA,
            pltpu.SemaphoreType.DMA,
            pltpu.SemaphoreType.DMA,
            pltpu.SemaphoreType.DMA,
            pltpu.SemaphoreType.DMA,
        ],
    )
    return f(srcp, dstp, aep, asrc, adst, *hs)


def _k4_body(*refs):
    o_refs = refs[:NG]
    q_ref, b_ref = refs[NG:NG + 2]
    out_ref = refs[NG + 2]
    bestv, brow = refs[NG + 3:]

    i = pl.program_id(0)
    nblk = pl.num_programs(0)
    q = q_ref[...]
    qn = q / jnp.maximum(jnp.sqrt(jnp.sum(q * q)), 1e-8)
    obs = []
    dots = jnp.zeros((NB, 1), F32)
    nn = jnp.zeros((NB, 1), F32)
    for g in range(NG):
        ob = o_refs[g][...] + b_ref[:, G * g:G * (g + 1)]
        obs.append(ob)
        dots = dots + jnp.sum(ob * qn[:, G * g:G * (g + 1)], axis=1,
                              keepdims=True)
        nn = nn + jnp.sum(ob * ob, axis=1, keepdims=True)
    sim = dots / jnp.maximum(jnp.sqrt(nn), 1e-8)
    m = jnp.max(sim)
    iota = lax.broadcasted_iota(jnp.int32, sim.shape, 0)
    il = jnp.min(jnp.where(sim == m, iota, jnp.int32(2**30)))

    @pl.when((i == 0) | (m > bestv[0]))
    def _():
        bestv[0] = m
        for g in range(NG):
            sel = jnp.where(iota == il, obs[g], 0.0)
            brow[:, pl.ds(G * g, G)] = jnp.sum(sel, axis=0, keepdims=True)

    @pl.when(i == nblk - 1)
    def _():
        out_ref[...] = brow[...]


def _k4(outs, q, bias):
    nblk = N // NB
    return pl.pallas_call(
        _k4_body,
        grid=(nblk,),
        in_specs=(
            [pl.BlockSpec((NB, G), lambda i: (i, 0)) for _ in range(NG)]
            + [pl.BlockSpec((1, D), lambda i: (0, 0)),
               pl.BlockSpec((1, D), lambda i: (0, 0))]
        ),
        out_specs=pl.BlockSpec((1, D), lambda i: (0, 0)),
        out_shape=jax.ShapeDtypeStruct((1, D), F32),
        scratch_shapes=[
            pltpu.SMEM((1,), F32),
            pltpu.VMEM((1, D), F32),
        ],
    )(*outs, q.reshape(1, D), bias.reshape(1, D))


def kernel(query_emb, edge_index, edge_attr, node_emb, W, att_src, att_dst,
           W_e, att_edge, bias):
    src = edge_index[0].reshape(NS, EPT)
    dst = edge_index[1].reshape(NS, EPT)
    srcp = jnp.pad(src, ((0, 0), (0, EPAD - EPT))).reshape(NS, NCHP, K)
    dstp = jnp.pad(dst, ((0, 0), (0, EPAD - EPT))).reshape(NS, NCHP, K)

    k1 = _k1(node_emb, W, att_src, att_dst, W_e, att_edge)
    hs = [jnp.pad(h, ((0, NP - N), (0, 0))) for h in k1[:NG]]
    asrc2, adst2, ve = k1[NG:]
    asrcp = jnp.pad(asrc2.reshape(N), (0, NP - N))
    adstp = jnp.pad(adst2.reshape(N), (0, NP - N))
    ae2 = _k2(edge_attr, ve)
    aep = jnp.pad(ae2.reshape(NS, EPT), ((0, 0), (0, EPAD - EPT))
                  ).reshape(NS, NCHP, K)
    outs = _k3(srcp, dstp, aep, asrcp, adstp, hs)
    res = _k4([o[:N] for o in outs], query_emb, bias)
    return res.reshape(D)


# speedup vs baseline: 1.0609x; 1.0146x over previous
import jax
import jax.numpy as jnp
from jax import lax
from jax.experimental import pallas as pl
from jax.experimental.pallas import tpu as pltpu
from jax.experimental.pallas import tpu_sc as plsc

N = 10000
E = 160000
D = 384
G = 64
NG = D // G
NS = 16
EPT = E // NS
K = 64
NCH = (EPT + K - 1) // K
NCHP = 160
EPAD = NCHP * K
NP = 10240
NRT = 640
NB = 1000
EB = 4000
F32 = jnp.float32


def _k1_body(x_ref, w_ref, as_ref, ad_ref, we_ref, aee_ref, *outs):
    h_refs = outs[:NG]
    s_ref, d_ref, ve_ref = outs[NG:]
    i = pl.program_id(0)
    h = jnp.dot(x_ref[...], w_ref[...], preferred_element_type=F32)
    for g in range(NG):
        h_refs[g][...] = h[:, G * g:G * (g + 1)]
    s_ref[...] = jnp.dot(h, as_ref[0, :], preferred_element_type=F32)[None, None, :]
    d_ref[...] = jnp.dot(h, ad_ref[0, :], preferred_element_type=F32)[None, None, :]

    @pl.when(i == 0)
    def _():
        ve_ref[...] = jnp.dot(we_ref[...], aee_ref[0, :],
                              preferred_element_type=F32)[None, :]


def _k1(node_emb, W, att_src, att_dst, W_e, att_edge):
    nblk = N // NB
    return pl.pallas_call(
        _k1_body,
        grid=(nblk,),
        in_specs=[
            pl.BlockSpec((NB, D), lambda i: (i, 0)),
            pl.BlockSpec((D, D), lambda i: (0, 0)),
            pl.BlockSpec((1, D), lambda i: (0, 0)),
            pl.BlockSpec((1, D), lambda i: (0, 0)),
            pl.BlockSpec((D, D), lambda i: (0, 0)),
            pl.BlockSpec((1, D), lambda i: (0, 0)),
        ],
        out_specs=(
            [pl.BlockSpec((NB, G), lambda i: (i, 0)) for _ in range(NG)]
            + [pl.BlockSpec((1, 1, NB), lambda i: (i, 0, 0)),
               pl.BlockSpec((1, 1, NB), lambda i: (i, 0, 0)),
               pl.BlockSpec((1, D), lambda i: (0, 0))]
        ),
        out_shape=(
            [jax.ShapeDtypeStruct((N, G), F32) for _ in range(NG)]
            + [jax.ShapeDtypeStruct((nblk, 1, NB), F32),
               jax.ShapeDtypeStruct((nblk, 1, NB), F32),
               jax.ShapeDtypeStruct((1, D), F32)]
        ),
    )(node_emb, W, att_src.reshape(1, D), att_dst.reshape(1, D),
      W_e, att_edge.reshape(1, D))


def _k2_body(ea_ref, ve_ref, ae_ref):
    ae_ref[...] = jnp.sum(ea_ref[...] * ve_ref[...], axis=1)[None, None, :]


def _k2(edge_attr, ve):
    eblk = E // EB
    return pl.pallas_call(
        _k2_body,
        grid=(eblk,),
        in_specs=[
            pl.BlockSpec((EB, D), lambda i: (i, 0)),
            pl.BlockSpec((1, D), lambda i: (0, 0)),
        ],
        out_specs=pl.BlockSpec((1, 1, EB), lambda i: (i, 0, 0)),
        out_shape=jax.ShapeDtypeStruct((eblk, 1, EB), F32),
    )(edge_attr, ve)


def _sc_body(*refs):
    (srcp, dstp, aep, asrc_h, adst_h) = refs[:5]
    h_hs = refs[5:5 + NG]
    out_hs = refs[5 + NG:5 + 2 * NG]
    (asrc_v, adst_v, w2_v, degt_v, saet_v, st_v, rows0, rows1, rows2,
     dst2_v, srcr, aer, cmax_v, el_v, rr_v, mx_v, iota2_v,
     acc_g, degg_g, saeg_g, sg_g, cmax_g,
     semA, semG0, semG1, semG2, semS0, semS1, semS2) = refs[5 + 2 * NG:]

    cid = lax.axis_index("c")
    sid = lax.axis_index("s")
    zeros16 = jnp.zeros((16,), F32)
    ones16 = jnp.ones((16,), F32)
    nsub = EPT // 16

    def _zero_rows0():
        def _z(r, _):
            for t in range(G // 16):
                rows0[r, pl.ds(16 * t, 16)] = zeros16
            return 0
        lax.fori_loop(0, K, _z, 0)

    _zero_rows0()

    def _z_tab(c, _):
        degt_v[c, pl.ds(0, 16)] = zeros16
        saet_v[c, pl.ds(0, 16)] = zeros16
        st_v[c, pl.ds(0, 16)] = zeros16
        return 0
    lax.fori_loop(0, NRT, _z_tab, 0)
    for t in range((EPAD - EPT) // 16):
        w2_v[pl.ds(EPT + 16 * t, 16)] = zeros16

    def _bi(c, _):
        iota2_v[c // 8, pl.ds(16 * (c % 8), 16)] = (
            lax.broadcasted_iota(jnp.int32, (16,), 0) + 16 * c)
        return 0
    lax.fori_loop(0, NRT // 16, _bi, 0)

    @pl.when(sid == 0)
    def _():
        pltpu.sync_copy(degt_v, degg_g)
        pltpu.sync_copy(saet_v, saeg_g)
        pltpu.sync_copy(st_v, sg_g)

    pltpu.sync_copy(asrc_h, asrc_v)
    pltpu.sync_copy(adst_h, adst_v)
    pltpu.sync_copy(dstp.at[sid], dst2_v)
    plsc.subcore_barrier()

    mx_v[...] = jnp.full((16,), -3e38, F32)

    def _fetch1(g, slot):
        pltpu.async_copy(srcp.at[sid, pl.ds(4 * g, 4)],
                         srcr.at[pl.ds(4 * slot, 4)], semA)
        pltpu.async_copy(aep.at[sid, pl.ds(4 * g, 4)],
                         aer.at[pl.ds(4 * slot, 4)], semA)

    def _wait1(slot):
        pltpu.make_async_copy(srcp.at[0, pl.ds(0, 4)],
                              srcr.at[pl.ds(4 * slot, 4)], semA).wait()
        pltpu.make_async_copy(aep.at[0, pl.ds(0, 4)],
                              aer.at[pl.ds(4 * slot, 4)], semA).wait()

    def _proc1(g, slot):
        def _sub(k, _):
            c = 16 * g + k

            @pl.when(c < nsub)
            def _():
                lr = 4 * slot + k // 4
                col = (k % 4) * 16
                jr = c // 4
                jc = (c % 4) * 16
                s16 = srcr[lr, pl.ds(col, 16)]
                d16 = dst2_v[jr, pl.ds(jc, 16)]
                a1 = plsc.load_gather(asrc_v, [s16])
                a2 = plsc.load_gather(adst_v, [d16])
                ae16 = aer[lr, pl.ds(col, 16)]
                al = a1 + a2 + ae16
                al = jnp.where(al >= 0.0, al, al * 0.2)
                w2_v[pl.ds(16 * c, 16)] = al
                dr16 = lax.shift_right_logical(d16, 4)
                dc16 = lax.bitwise_and(d16, jnp.int32(15))
                plsc.addupdate_scatter(degt_v, [dr16, dc16], ones16)
                plsc.addupdate_scatter(saet_v, [dr16, dc16], ae16)
                mx_v[...] = jnp.maximum(mx_v[...], al)
            return 0
        lax.fori_loop(0, 16, _sub, 0)

    ngrp = NCHP // 4
    _fetch1(0, 0)
    _fetch1(1, 1)

    def _p1(gg, _):
        g0 = 2 * gg
        _wait1(0)
        _proc1(g0, 0)

        @pl.when(g0 + 2 < ngrp)
        def _():
            _fetch1(g0 + 2, 0)
        _wait1(1)
        _proc1(g0 + 1, 1)

        @pl.when(g0 + 3 < ngrp)
        def _():
            _fetch1(g0 + 3, 1)
        return 0
    lax.fori_loop(0, ngrp // 2, _p1, 0)

    el_v[...] = jnp.full((16,), jnp.max(mx_v[...]), F32)
    pltpu.sync_copy(el_v, cmax_g.at[sid])

    for r in range(NRT // 128):
        sl = pl.ds(128 * r, 128)
        pltpu.sync_copy(degt_v.at[sl], degg_g.at[iota2_v.at[r]], add=True)
        pltpu.sync_copy(saet_v.at[sl], saeg_g.at[iota2_v.at[r]], add=True)
    plsc.subcore_barrier()

    pltpu.sync_copy(cmax_g, cmax_v)
    cm = jnp.full((16,), -3e38, F32)
    for i in range(NS):
        cm = jnp.maximum(cm, cmax_v[i, pl.ds(0, 16)])
    cglob = jnp.max(cm)

    def _p2(c, _):
        jr = c // 4
        jc = (c % 4) * 16
        d16 = dst2_v[jr, pl.ds(jc, 16)]
        w16 = jnp.exp(w2_v[pl.ds(16 * c, 16)] - cglob)
        w2_v[pl.ds(16 * c, 16)] = w16
        dr16 = lax.shift_right_logical(d16, 4)
        dc16 = lax.bitwise_and(d16, jnp.int32(15))
        plsc.addupdate_scatter(st_v, [dr16, dc16], w16)
        return 0
    lax.fori_loop(0, nsub, _p2, 0)

    for r in range(NRT // 128):
        pltpu.sync_copy(st_v.at[pl.ds(128 * r, 128)],
                        sg_g.at[iota2_v.at[r]], add=True)
    plsc.subcore_barrier()

    pltpu.sync_copy(degg_g, degt_v)
    pltpu.sync_copy(saeg_g, saet_v)
    pltpu.sync_copy(sg_g, st_v)
    plsc.subcore_barrier()

    def _pre(q, _):
        nc = 40 * sid + q
        b = 16 * nc
        deg16 = degt_v[nc, pl.ds(0, 16)]
        sae16 = saet_v[nc, pl.ds(0, 16)]
        s16 = st_v[nc, pl.ds(0, 16)]
        al = (asrc_v[pl.ds(b, 16)] + adst_v[pl.ds(b, 16)]
              + sae16 / jnp.maximum(deg16, 1.0))
        al = jnp.where(al >= 0.0, al, al * 0.2)
        el16 = jnp.exp(al - cglob)
        r16 = 1.0 / (s16 + el16 + 1e-16)
        asrc_v[pl.ds(b, 16)] = el16
        adst_v[pl.ds(b, 16)] = r16
        return 0
    lax.fori_loop(0, NP // (16 * NS), _pre, 0)

    def _group(h_h, out_h, first):
        if first:
            _zero_rows0()

            def _z_acc(t, _):
                blk = sid + NS * t
                pltpu.sync_copy(rows0, acc_g.at[pl.ds(K * blk, K)])
                return 0
            lax.fori_loop(0, NP // (NS * K), _z_acc, 0)
            plsc.subcore_barrier()

        def _fetchrow(j, slot):
            pltpu.sync_copy(srcp.at[sid, j], srcr.at[slot])

        def _gather(buf, sem, slot):
            pltpu.async_copy(h_h.at[srcr.at[slot]], buf, sem)

        def _wait_g(buf, sem, slot):
            pltpu.make_async_copy(h_h.at[srcr.at[slot]], buf, sem).wait()

        def _proc4(buf, j):
            @plsc.parallel_loop(0, K, step=1, unroll=4)
            def _prow(u):
                wv = plsc.load_gather(
                    w2_v, [jnp.full((16,), K * j + u, jnp.int32)])
                for t in range(G // 16):
                    sl = pl.ds(16 * t, 16)
                    buf[u, sl] = buf[u, sl] * wv

        def _scat(buf, sem, j):
            pltpu.async_copy(buf, acc_g.at[dst2_v.at[j]], sem, add=True)

        def _wait_s(buf, sem):
            pltpu.make_async_copy(buf, acc_g.at[dst2_v.at[0]], sem).wait()

        bufs = (rows0, rows1, rows2)
        semsG = (semG0, semG1, semG2)
        semsS = (semS0, semS1, semS2)
        for p in range(3):
            _fetchrow(p, p)
            _gather(bufs[p], semsG[p], p)

        def _p4(kk, _):
            for p in range(3):
                j = 3 * kk + p
                _wait_g(bufs[p], semsG[p], p)
                _proc4(bufs[p], j)
                _scat(bufs[p], semsS[p], j)

                @pl.when(j + 3 < NCH)
                def _(j=j, p=p):
                    _wait_s(bufs[p], semsS[p])
                    _fetchrow(j + 3, p)
                    _gather(bufs[p], semsG[p], p)
            return 0
        lax.fori_loop(0, NCH // 3, _p4, 0)
        _wait_g(rows0, semG0, 0)
        _proc4(rows0, NCH - 1)
        _scat(rows0, semS0, NCH - 1)
        _wait_s(rows0, semS0)
        _wait_s(rows1, semS1)
        _wait_s(rows2, semS2)

        iota16 = lax.broadcasted_iota(jnp.int32, (16,), 0)

        def _self_idx(slot, rb):
            for t in range(4):
                srcr[slot, pl.ds(16 * t, 16)] = iota16 + (rb + 16 * t)

        def _proc4b(buf, rb):
            @plsc.parallel_loop(0, K, step=1, unroll=4)
            def _prow(u):
                wv = plsc.load_gather(
                    asrc_v, [jnp.full((16,), rb + u, jnp.int32)])
                for t in range(G // 16):
                    sl = pl.ds(16 * t, 16)
                    buf[u, sl] = buf[u, sl] * wv

        rb0 = 640 * sid
        pltpu.async_copy(h_h.at[pl.ds(rb0, K)], rows0, semG0)
        pltpu.async_copy(h_h.at[pl.ds(rb0 + K, K)], rows1, semG1)

        def _p4b(bb, _):
            b0 = 2 * bb
            rbA = rb0 + K * b0
            pltpu.make_async_copy(h_h.at[pl.ds(0, K)], rows0, semG0).wait()
            _proc4b(rows0, rbA)
            _self_idx(0, rbA)
            pltpu.async_copy(rows0, acc_g.at[srcr.at[0]], semS0, add=True)

            @pl.when(b0 + 2 < 10)
            def _():
                _wait_s(rows0, semS0)
                pltpu.async_copy(h_h.at[pl.ds(rbA + 2 * K, K)], rows0, semG0)
            pltpu.make_async_copy(h_h.at[pl.ds(0, K)], rows1, semG1).wait()
            _proc4b(rows1, rbA + K)
            _self_idx(1, rbA + K)
            pltpu.async_copy(rows1, acc_g.at[srcr.at[1]], semS1, add=True)

            @pl.when(b0 + 3 < 10)
            def _():
                _wait_s(rows1, semS1)
                pltpu.async_copy(h_h.at[pl.ds(rbA + 3 * K, K)], rows1, semG1)
            return 0
        lax.fori_loop(0, 5, _p4b, 0)
        _wait_s(rows0, semS0)
        _wait_s(rows1, semS1)
        plsc.subcore_barrier()

        _zero_rows0()
        rb0 = 640 * sid
        pltpu.async_copy(acc_g.at[pl.ds(rb0, K)], rows1, semG0)
        pltpu.async_copy(acc_g.at[pl.ds(rb0 + K, K)], rows2, semG1)

        def _p5core(buf, rb, semg, sems):
            pltpu.make_async_copy(acc_g.at[pl.ds(0, K)], buf, semg).wait()
            pltpu.async_copy(rows0, acc_g.at[pl.ds(rb, K)], semA)

            @plsc.parallel_loop(0, K, step=1, unroll=4)
            def _prow(u):
                rv = plsc.load_gather(
                    adst_v, [jnp.full((16,), rb + u, jnp.int32)])
                for t2 in range(G // 16):
                    sl = pl.ds(16 * t2, 16)
                    buf[u, sl] = buf[u, sl] * rv
            pltpu.async_copy(buf, out_h.at[pl.ds(rb, K)], sems)

        def _p5(bb, _):
            b0 = 2 * bb
            rbA = rb0 + K * b0
            _p5core(rows1, rbA, semG0, semS0)

            @pl.when(b0 + 2 < 10)
            def _():
                pltpu.make_async_copy(rows1, out_h.at[pl.ds(0, K)],
                                      semS0).wait()
                pltpu.async_copy(acc_g.at[pl.ds(rbA + 2 * K, K)], rows1,
                                 semG0)
            _p5core(rows2, rbA + K, semG1, semS1)

            @pl.when(b0 + 3 < 10)
            def _():
                pltpu.make_async_copy(rows2, out_h.at[pl.ds(0, K)],
                                      semS1).wait()
                pltpu.async_copy(acc_g.at[pl.ds(rbA + 3 * K, K)], rows2,
                                 semG1)
            return 0
        lax.fori_loop(0, 5, _p5, 0)
        pltpu.make_async_copy(rows1, out_h.at[pl.ds(0, K)], semS0).wait()
        pltpu.make_async_copy(rows2, out_h.at[pl.ds(0, K)], semS1).wait()
        for _d in range(10):
            pltpu.make_async_copy(rows0, acc_g.at[pl.ds(0, K)], semA).wait()
        plsc.subcore_barrier()

    for g in range(NG // 2):
        @pl.when(cid == 0)
        def _(g=g):
            _group(h_hs[g], out_hs[g], g == 0)

        @pl.when(cid == 1)
        def _(g=g):
            _group(h_hs[NG // 2 + g], out_hs[NG // 2 + g], g == 0)


def _k3(srcp, dstp, aep, asrc, adst, hs):
    mesh = plsc.VectorSubcoreMesh(core_axis_name="c", subcore_axis_name="s",
                                  num_cores=2, num_subcores=NS)
    f = pl.kernel(
        _sc_body,
        out_type=[jax.ShapeDtypeStruct((NP, G), F32) for _ in range(NG)],
        mesh=mesh,
        compiler_params=pltpu.CompilerParams(needs_layout_passes=False,
                                             use_tc_tiling_on_sc=False),
        scratch_types=[
            pltpu.VMEM((NP,), F32),
            pltpu.VMEM((NP,), F32),
            pltpu.VMEM((EPAD,), F32),
            pltpu.VMEM((NRT, 16), F32),
            pltpu.VMEM((NRT, 16), F32),
            pltpu.VMEM((NRT, 16), F32),
            pltpu.VMEM((K, G), F32),
            pltpu.VMEM((K, G), F32),
            pltpu.VMEM((K, G), F32),
            pltpu.VMEM((NCHP, K), jnp.int32),
            pltpu.VMEM((8, K), jnp.int32),
            pltpu.VMEM((8, K), F32),
            pltpu.VMEM((NS, 16), F32),
            pltpu.VMEM((16,), F32),
            pltpu.VMEM((16,), F32),
            pltpu.VMEM((16,), F32),
            pltpu.VMEM((5, 128), jnp.int32),
            pltpu.VMEM_SHARED((NP, G), F32),
            pltpu.VMEM_SHARED((NRT, 16), F32),
            pltpu.VMEM_SHARED((NRT, 16), F32),
            pltpu.VMEM_SHARED((NRT, 16), F32),
            pltpu.VMEM_SHARED((NS, 16), F32),
            pltpu.SemaphoreType.DMA,
            pltpu.SemaphoreType.DMA,
            pltpu.SemaphoreType.DMA,
            pltpu.SemaphoreType.DMA,
            pltpu.SemaphoreType.DMA,
            pltpu.SemaphoreType.DMA,
            pltpu.SemaphoreType.DMA,
        ],
    )
    return f(srcp, dstp, aep, asrc, adst, *hs)


def _k4_body(*refs):
    o_refs = refs[:NG]
    q_ref, b_ref = refs[NG:NG + 2]
    out_ref = refs[NG + 2]
    bestv, brow = refs[NG + 3:]

    i = pl.program_id(0)
    nblk = pl.num_programs(0)
    q = q_ref[...]
    qn = q / jnp.maximum(jnp.sqrt(jnp.sum(q * q)), 1e-8)
    obs = []
    dots = jnp.zeros((NB, 1), F32)
    nn = jnp.zeros((NB, 1), F32)
    for g in range(NG):
        ob = o_refs[g][...] + b_ref[:, G * g:G * (g + 1)]
        obs.append(ob)
        dots = dots + jnp.sum(ob * qn[:, G * g:G * (g + 1)], axis=1,
                              keepdims=True)
        nn = nn + jnp.sum(ob * ob, axis=1, keepdims=True)
    sim = dots / jnp.maximum(jnp.sqrt(nn), 1e-8)
    m = jnp.max(sim)
    iota = lax.broadcasted_iota(jnp.int32, sim.shape, 0)
    il = jnp.min(jnp.where(sim == m, iota, jnp.int32(2**30)))

    @pl.when((i == 0) | (m > bestv[0]))
    def _():
        bestv[0] = m
        for g in range(NG):
            sel = jnp.where(iota == il, obs[g], 0.0)
            brow[:, pl.ds(G * g, G)] = jnp.sum(sel, axis=0, keepdims=True)

    @pl.when(i == nblk - 1)
    def _():
        out_ref[...] = brow[...]


def _k4(outs, q, bias):
    nblk = N // NB
    return pl.pallas_call(
        _k4_body,
        grid=(nblk,),
        in_specs=(
            [pl.BlockSpec((NB, G), lambda i: (i, 0)) for _ in range(NG)]
            + [pl.BlockSpec((1, D), lambda i: (0, 0)),
               pl.BlockSpec((1, D), lambda i: (0, 0))]
        ),
        out_specs=pl.BlockSpec((1, D), lambda i: (0, 0)),
        out_shape=jax.ShapeDtypeStruct((1, D), F32),
        scratch_shapes=[
            pltpu.SMEM((1,), F32),
            pltpu.VMEM((1, D), F32),
        ],
    )(*outs, q.reshape(1, D), bias.reshape(1, D))


def kernel(query_emb, edge_index, edge_attr, node_emb, W, att_src, att_dst,
           W_e, att_edge, bias):
    src = edge_index[0].reshape(NS, EPT)
    dst = edge_index[1].reshape(NS, EPT)
    srcp = jnp.pad(src, ((0, 0), (0, EPAD - EPT))).reshape(NS, NCHP, K)
    dstp = jnp.pad(dst, ((0, 0), (0, EPAD - EPT))).reshape(NS, NCHP, K)

    k1 = _k1(node_emb, W, att_src, att_dst, W_e, att_edge)
    hs = [jnp.pad(h, ((0, NP - N), (0, 0))) for h in k1[:NG]]
    asrc2, adst2, ve = k1[NG:]
    asrcp = jnp.pad(asrc2.reshape(N), (0, NP - N))
    adstp = jnp.pad(adst2.reshape(N), (0, NP - N))
    ae2 = _k2(edge_attr, ve)
    aep = jnp.pad(ae2.reshape(NS, EPT), ((0, 0), (0, EPAD - EPT))
                  ).reshape(NS, NCHP, K)
    outs = _k3(srcp, dstp, aep, asrcp, adstp, hs)
    res = _k4([o[:N] for o in outs], query_emb, bias)
    return res.reshape(D)
